# baseline (speedup 1.0000x reference)
import functools

import jax
import jax.numpy as jnp
from jax import lax
from jax.experimental import pallas as pl
from jax.experimental.pallas import tpu as pltpu
from jax.experimental.pallas import tpu_sc as plsc

_NUM_WORKERS = 32
_CHUNK_ROWS = 8
_LANES = 16
_UNROLL = 4
_XDEPTH = 3
_EDEPTH = 2


def kernel(x, emb_weight):
    batch, seq_len, d = x.shape
    total_rows = batch * seq_len
    seq_per_w = seq_len // _NUM_WORKERS
    C = _CHUNK_ROWS
    n_chunks = seq_per_w // C
    chunk = C * d
    groups = chunk // (_LANES * _UNROLL)

    x2 = x.reshape(total_rows, d)
    dummy = jnp.zeros((chunk,), jnp.float32)

    mesh = plsc.VectorSubcoreMesh(core_axis_name="c", subcore_axis_name="s")

    vmem_types = [pltpu.VMEM((chunk,), jnp.float32)
                  for _ in range(_XDEPTH * batch + _EDEPTH)]
    sem_types = [pltpu.SemaphoreType.DMA
                 for _ in range(2 * _XDEPTH + _EDEPTH)]

    @functools.partial(
        pl.kernel,
        mesh=mesh,
        out_type=jax.ShapeDtypeStruct((total_rows, d), jnp.float32),
        scratch_types=vmem_types + sem_types,
    )
    def sc_add(x_hbm, emb_hbm, dummy_hbm, out_hbm, *scr):
        bufs = scr[:_XDEPTH * batch + _EDEPTH]
        sems = scr[_XDEPTH * batch + _EDEPTH:]
        xg = tuple(tuple(bufs[p * batch + j] for j in range(batch))
                   for p in range(_XDEPTH))
        eb = tuple(bufs[_XDEPTH * batch + e] for e in range(_EDEPTH))
        sx = sems[:_XDEPTH]
        so = sems[_XDEPTH:2 * _XDEPTH]
        se = sems[2 * _XDEPTH:]

        cid = lax.axis_index("c")
        sid = lax.axis_index("s")
        wid = sid * 2 + cid
        seq_base = wid * seq_per_w

        def issue_in(k, p, e):
            erow = seq_base + k * C

            def cp_body(r, _):
                dst = pl.ds(r * d, d)
                pltpu.async_copy(emb_hbm.at[erow + r], eb[e].at[dst], se[e])
                for j in range(batch):
                    pltpu.async_copy(x_hbm.at[j * seq_len + erow + r],
                                     xg[p][j].at[dst], sx[p])
                return 0

            lax.fori_loop(0, C, cp_body, 0)

        def wait_in(p, e):
            pltpu.make_async_copy(dummy_hbm, eb[e], se[e]).wait()
            for j in range(batch):
                pltpu.make_async_copy(dummy_hbm, xg[p][j], sx[p]).wait()

        def wait_out(p):
            for j in range(batch):
                pltpu.make_async_copy(xg[p][j], dummy_hbm, so[p]).wait()

        def compute(p, e):
            def add_body(i, _):
                base = i * (_LANES * _UNROLL)
                for u in range(_UNROLL):
                    sl = pl.ds(base + u * _LANES, _LANES)
                    ve = eb[e][sl]
                    for j in range(batch):
                        plsc.addupdate(xg[p][j].at[sl], ve)
                return 0

            lax.fori_loop(0, groups, add_body, 0)

        def issue_out(k, p):
            erow = seq_base + k * C

            def cp_body(r, _):
                src = pl.ds(r * d, d)
                for j in range(batch):
                    pltpu.async_copy(xg[p][j].at[src],
                                     out_hbm.at[j * seq_len + erow + r], so[p])
                return 0

            lax.fori_loop(0, C, cp_body, 0)

        def step(k, kmod3, kmod2, reuse_wait, issue_next):
            wait_in(kmod3, kmod2)
            compute(kmod3, kmod2)
            issue_out(k, kmod3)
            if issue_next:
                p_next = (kmod3 + 2) % _XDEPTH
                if reuse_wait:
                    wait_out(p_next)
                issue_in(k + 2, p_next, kmod2)

        issue_in(0, 0, 0)
        issue_in(1, 1, 1)
        step(0, 0, 0, False, True)
        step(1, 1, 1, True, True)

        def six_body(i, _):
            k0 = 2 + 6 * i
            for jj in range(6):
                step(k0 + jj, (2 + jj) % _XDEPTH, jj % _EDEPTH, True, True)
            return 0

        lax.fori_loop(0, (n_chunks - 4) // 6, six_body, 0)

        step(n_chunks - 2, (n_chunks - 2) % _XDEPTH, (n_chunks - 2) % _EDEPTH,
             False, False)
        step(n_chunks - 1, (n_chunks - 1) % _XDEPTH, (n_chunks - 1) % _EDEPTH,
             False, False)
        wait_out((n_chunks - 3) % _XDEPTH)
        wait_out((n_chunks - 2) % _XDEPTH)
        wait_out((n_chunks - 1) % _XDEPTH)

    out2 = sc_add(x2, emb_weight, dummy)
    return out2.reshape(batch, seq_len, d)


# speedup vs baseline: 1.0373x; 1.0373x over previous
import functools

import jax
import jax.numpy as jnp
from jax import lax
from jax.experimental import pallas as pl
from jax.experimental.pallas import tpu as pltpu
from jax.experimental.pallas import tpu_sc as plsc

_NUM_WORKERS = 32
_CHUNK_ROWS = 8
_LANES = 16
_UNROLL = 4
_DEPTH = 3


def kernel(x, emb_weight):
    batch, seq_len, d = x.shape
    total_rows = batch * seq_len
    seq_per_w = seq_len // _NUM_WORKERS
    C = _CHUNK_ROWS
    n_chunks = seq_per_w // C
    chunk = C * d
    groups = chunk // (_LANES * _UNROLL)

    x2 = x.reshape(total_rows, d)
    dummy = jnp.zeros((chunk,), jnp.float32)

    mesh = plsc.VectorSubcoreMesh(core_axis_name="c", subcore_axis_name="s")

    vmem_types = [pltpu.VMEM((chunk,), jnp.float32)
                  for _ in range(_DEPTH * (batch + 1))]
    sem_types = [pltpu.SemaphoreType.DMA for _ in range(3 * _DEPTH)]

    @functools.partial(
        pl.kernel,
        mesh=mesh,
        out_type=jax.ShapeDtypeStruct((total_rows, d), jnp.float32),
        scratch_types=vmem_types + sem_types,
    )
    def sc_add(x_hbm, emb_hbm, dummy_hbm, out_hbm, *scr):
        bufs = scr[:_DEPTH * (batch + 1)]
        sems = scr[_DEPTH * (batch + 1):]
        xg = tuple(tuple(bufs[p * batch + j] for j in range(batch))
                   for p in range(_DEPTH))
        eb = tuple(bufs[_DEPTH * batch + p] for p in range(_DEPTH))
        sx = sems[:_DEPTH]
        so = sems[_DEPTH:2 * _DEPTH]
        se = sems[2 * _DEPTH:]

        cid = lax.axis_index("c")
        sid = lax.axis_index("s")
        wid = sid * 2 + cid
        seq_base = wid * seq_per_w

        def issue_in(k, p):
            erow = seq_base + k * C

            def cp_body(r, _):
                dst = pl.ds(r * d, d)
                pltpu.async_copy(emb_hbm.at[erow + r], eb[p].at[dst], se[p])
                for j in range(batch):
                    pltpu.async_copy(x_hbm.at[j * seq_len + erow + r],
                                     xg[p][j].at[dst], sx[p])
                return 0

            lax.fori_loop(0, C, cp_body, 0)

        def wait_in(p):
            pltpu.make_async_copy(dummy_hbm, eb[p], se[p]).wait()
            for j in range(batch):
                pltpu.make_async_copy(dummy_hbm, xg[p][j], sx[p]).wait()

        def wait_out(p):
            for j in range(batch):
                pltpu.make_async_copy(xg[p][j], dummy_hbm, so[p]).wait()

        def compute(p):
            def add_body(i, _):
                base = i * (_LANES * _UNROLL)
                for u in range(_UNROLL):
                    sl = pl.ds(base + u * _LANES, _LANES)
                    ve = eb[p][sl]
                    for j in range(batch):
                        plsc.addupdate(xg[p][j].at[sl], ve)
                return 0

            lax.fori_loop(0, groups, add_body, 0)

        def issue_out(k, p):
            erow = seq_base + k * C

            def cp_body(r, _):
                src = pl.ds(r * d, d)
                for j in range(batch):
                    pltpu.async_copy(xg[p][j].at[src],
                                     out_hbm.at[j * seq_len + erow + r], so[p])
                return 0

            lax.fori_loop(0, C, cp_body, 0)

        def step(k, p, first, last):
            wait_in(p)
            if not last:
                p_next = (p + 2) % _DEPTH
                if not first:
                    wait_out(p_next)
                issue_in(k + 2, p_next)
            compute(p)
            issue_out(k, p)

        issue_in(0, 0)
        issue_in(1, 1)
        step(0, 0, True, False)
        step(1, 1, False, False)

        def tri_body(i, _):
            k0 = 2 + 3 * i
            for jj in range(3):
                step(k0 + jj, (2 + jj) % _DEPTH, False, False)
            return 0

        lax.fori_loop(0, (n_chunks - 4) // 3, tri_body, 0)

        step(n_chunks - 2, (n_chunks - 2) % _DEPTH, False, True)
        step(n_chunks - 1, (n_chunks - 1) % _DEPTH, False, True)
        wait_out((n_chunks - 3) % _DEPTH)
        wait_out((n_chunks - 2) % _DEPTH)
        wait_out((n_chunks - 1) % _DEPTH)

    out2 = sc_add(x2, emb_weight, dummy)
    return out2.reshape(batch, seq_len, d)


# speedup vs baseline: 1.0475x; 1.0099x over previous
import functools

import jax
import jax.numpy as jnp
from jax import lax
from jax.experimental import pallas as pl
from jax.experimental.pallas import tpu as pltpu
from jax.experimental.pallas import tpu_sc as plsc

_NUM_WORKERS = 32
_CHUNK_ROWS = 4
_LANES = 16
_UNROLL = 4
_DEPTH = 4


def kernel(x, emb_weight):
    batch, seq_len, d = x.shape
    total_rows = batch * seq_len
    seq_per_w = seq_len // _NUM_WORKERS
    C = _CHUNK_ROWS
    n_chunks = seq_per_w // C
    chunk = C * d
    groups = chunk // (_LANES * _UNROLL)

    x2 = x.reshape(total_rows, d)
    dummy = jnp.zeros((chunk,), jnp.float32)

    mesh = plsc.VectorSubcoreMesh(core_axis_name="c", subcore_axis_name="s")

    vmem_types = [pltpu.VMEM((chunk,), jnp.float32)
                  for _ in range(_DEPTH * (batch + 1))]
    sem_types = [pltpu.SemaphoreType.DMA for _ in range(3 * _DEPTH)]

    @functools.partial(
        pl.kernel,
        mesh=mesh,
        out_type=jax.ShapeDtypeStruct((total_rows, d), jnp.float32),
        scratch_types=vmem_types + sem_types,
    )
    def sc_add(x_hbm, emb_hbm, dummy_hbm, out_hbm, *scr):
        bufs = scr[:_DEPTH * (batch + 1)]
        sems = scr[_DEPTH * (batch + 1):]
        xg = tuple(tuple(bufs[p * batch + j] for j in range(batch))
                   for p in range(_DEPTH))
        eb = tuple(bufs[_DEPTH * batch + p] for p in range(_DEPTH))
        sx = sems[:_DEPTH]
        so = sems[_DEPTH:2 * _DEPTH]
        se = sems[2 * _DEPTH:]

        cid = lax.axis_index("c")
        sid = lax.axis_index("s")
        wid = sid * 2 + cid
        seq_base = wid * seq_per_w

        def issue_in(k, p):
            erow = seq_base + k * C

            def cp_body(r, _):
                dst = pl.ds(r * d, d)
                pltpu.async_copy(emb_hbm.at[erow + r], eb[p].at[dst], se[p])
                for j in range(batch):
                    pltpu.async_copy(x_hbm.at[j * seq_len + erow + r],
                                     xg[p][j].at[dst], sx[p])
                return 0

            lax.fori_loop(0, C, cp_body, 0)

        def wait_in(p):
            pltpu.make_async_copy(dummy_hbm, eb[p], se[p]).wait()
            for j in range(batch):
                pltpu.make_async_copy(dummy_hbm, xg[p][j], sx[p]).wait()

        def wait_out(p):
            for j in range(batch):
                pltpu.make_async_copy(xg[p][j], dummy_hbm, so[p]).wait()

        def compute(p):
            def add_body(i, _):
                base = i * (_LANES * _UNROLL)
                for u in range(_UNROLL):
                    sl = pl.ds(base + u * _LANES, _LANES)
                    ve = eb[p][sl]
                    for j in range(batch):
                        plsc.addupdate(xg[p][j].at[sl], ve)
                return 0

            lax.fori_loop(0, groups, add_body, 0)

        def issue_out(k, p):
            erow = seq_base + k * C

            def cp_body(r, _):
                src = pl.ds(r * d, d)
                for j in range(batch):
                    pltpu.async_copy(xg[p][j].at[src],
                                     out_hbm.at[j * seq_len + erow + r], so[p])
                return 0

            lax.fori_loop(0, C, cp_body, 0)

        def step(k, p, first, last):
            wait_in(p)
            if not last:
                p_next = (p + 2) % _DEPTH
                if not first:
                    wait_out(p_next)
                issue_in(k + 2, p_next)
            compute(p)
            issue_out(k, p)

        issue_in(0, 0)
        issue_in(1, 1)
        step(0, 0, True, False)
        step(1, 1, True, False)

        def ring_body(i, _):
            k0 = 2 + _DEPTH * i
            for jj in range(_DEPTH):
                step(k0 + jj, (2 + jj) % _DEPTH, False, False)
            return 0

        lax.fori_loop(0, (n_chunks - 4) // _DEPTH, ring_body, 0)

        step(n_chunks - 2, (n_chunks - 2) % _DEPTH, False, True)
        step(n_chunks - 1, (n_chunks - 1) % _DEPTH, False, True)
        for t in range(_DEPTH):
            wait_out((n_chunks - _DEPTH + t) % _DEPTH)

    out2 = sc_add(x2, emb_weight, dummy)
    return out2.reshape(batch, seq_len, d)
